# baseline (speedup 1.0000x reference)
import functools

import jax
import jax.numpy as jnp
import numpy as np
from jax import lax
from jax.experimental import pallas as pl
from jax.experimental.pallas import tpu as pltpu
from jax.experimental.pallas import tpu_sc as plsc

_B = 64
_SPATIAL = 1024
_CDIM = 768
_SS = 32
_PS = 16
_NPOOL = 256
_TDIM = 32
_NC = 2
_NS = 16
_NW = _NC * _NS
_BLK_S = 64

_K_SC = 32
_SC_BLOCKS = _K_SC * (_SPATIAL // _BLK_S)
_BLK_PER_W = _SC_BLOCKS // _NW
assert _SC_BLOCKS % _NW == 0 and _BLK_PER_W % 2 == 0

_CH = np.linspace(0, _CDIM - 1, _TDIM).astype(np.int32)


def _issue_block(feat_hbm, blk, buf, sem):
    pltpu.async_copy(
        feat_hbm.at[pl.ds(blk * _BLK_S, _BLK_S), :],
        buf,
        sem,
    )


def _drain_block(feat_hbm, buf, sem):
    pltpu.make_async_copy(
        feat_hbm.at[pl.ds(0, _BLK_S), :],
        buf,
        sem,
    ).wait()


def _compute_block(ch_v, buf, out_v):
    for h in range(2):
        chv = ch_v[h]
        for c_col in range(_PS):
            acc = None
            for r in (2 * c_col, 2 * c_col + 1,
                      _SS + 2 * c_col, _SS + 2 * c_col + 1):
                rv = jnp.full((16,), r, jnp.int32)
                v = plsc.load_gather(buf, [rv, chv])
                acc = v if acc is None else acc + v
            out_v[c_col, pl.ds(h * 16, 16)] = acc * jnp.float32(0.25)


def _store_block(out_hbm, blk, out_v):
    b = blk // (_SPATIAL // _BLK_S)
    g = blk % (_SPATIAL // _BLK_S)
    pltpu.sync_copy(out_v, out_hbm.at[b, pl.ds(g * _PS, _PS)])


def _sc_body(feat_hbm, ch_hbm, out_hbm, ch_v, buf_a, buf_b, out_v,
             sem_a, sem_b):
    wid = lax.axis_index("s") * _NC + lax.axis_index("c")
    pltpu.sync_copy(ch_hbm, ch_v)

    blk0 = wid * _BLK_PER_W
    _issue_block(feat_hbm, blk0, buf_a, sem_a)

    def blk_body(g, _):
        blk = blk0 + g * 2
        _issue_block(feat_hbm, blk + 1, buf_b, sem_b)
        _drain_block(feat_hbm, buf_a, sem_a)
        _compute_block(ch_v, buf_a, out_v)
        _store_block(out_hbm, blk, out_v)

        @pl.when(g < _BLK_PER_W // 2 - 1)
        def _prefetch():
            _issue_block(feat_hbm, blk + 2, buf_a, sem_a)

        _drain_block(feat_hbm, buf_b, sem_b)
        _compute_block(ch_v, buf_b, out_v)
        _store_block(out_hbm, blk + 1, out_v)
        return _

    lax.fori_loop(0, _BLK_PER_W // 2, blk_body, None)


def _sc_part(features):
    feat2d = features.reshape(_B * _SPATIAL, _CDIM)
    chans = jnp.asarray(_CH.reshape(2, 16))
    sc_call = functools.partial(
        pl.kernel,
        mesh=plsc.VectorSubcoreMesh(core_axis_name="c", subcore_axis_name="s"),
        compiler_params=pltpu.CompilerParams(
            use_tc_tiling_on_sc=True, needs_layout_passes=False),
        out_type=jax.ShapeDtypeStruct((_K_SC, _NPOOL, _TDIM), jnp.float32),
        scratch_types=[
            pltpu.VMEM((2, 16), jnp.int32),
            pltpu.VMEM((_BLK_S, _CDIM), jnp.float32),
            pltpu.VMEM((_BLK_S, _CDIM), jnp.float32),
            pltpu.VMEM((_PS, _TDIM), jnp.float32),
            pltpu.SemaphoreType.DMA,
            pltpu.SemaphoreType.DMA,
        ],
    )(_sc_body)
    return sc_call(feat2d, chans)


def _pool_matrix() -> np.ndarray:
    P = np.zeros((_NPOOL, _SS * _SS), dtype=np.float32)
    for R in range(_PS):
        for C in range(_PS):
            p = R * _PS + C
            for dr in range(2):
                for dc in range(2):
                    s = (2 * R + dr) * _SS + (2 * C + dc)
                    P[p, s] = 0.25
    return P


def _select_matrix() -> np.ndarray:
    S = np.zeros((_CDIM, _TDIM), dtype=np.float32)
    S[_CH, np.arange(_TDIM)] = 1.0
    return S


def _tc_body(x_ref, p_ref, s_ref, o_ref):
    x = x_ref[0]
    sel = jnp.dot(x, s_ref[...], preferred_element_type=jnp.float32)
    o_ref[0] = jnp.dot(p_ref[...], sel, preferred_element_type=jnp.float32)


def _tc_part(features):
    nb = _B - _K_SC
    P = jnp.asarray(_pool_matrix())
    S = jnp.asarray(_select_matrix())
    return pl.pallas_call(
        _tc_body,
        grid=(nb,),
        in_specs=[
            pl.BlockSpec((1, _SPATIAL, _CDIM), lambda i: (i + _K_SC, 0, 0)),
            pl.BlockSpec((_NPOOL, _SPATIAL), lambda i: (0, 0)),
            pl.BlockSpec((_CDIM, _TDIM), lambda i: (0, 0)),
        ],
        out_specs=pl.BlockSpec((1, _NPOOL, _TDIM), lambda i: (i + _K_SC, 0, 0)),
        out_shape=jax.ShapeDtypeStruct((_B, _NPOOL, _TDIM), jnp.float32),
    )(features, P, S)


def kernel(features):
    sc_out = _sc_part(features)
    tc_out = _tc_part(features)
    return lax.dynamic_update_slice(tc_out, sc_out, (0, 0, 0))


# speedup vs baseline: 1.0427x; 1.0427x over previous
import functools

import jax
import jax.numpy as jnp
import numpy as np
from jax import lax
from jax.experimental import pallas as pl
from jax.experimental.pallas import tpu as pltpu
from jax.experimental.pallas import tpu_sc as plsc

_B = 64
_SPATIAL = 1024
_CDIM = 768
_SS = 32
_PS = 16
_NPOOL = 256
_TDIM = 32
_NC = 2
_NS = 16
_NW = _NC * _NS
_BLK_S = 64

_K_SC = 32
_SC_BLOCKS = _K_SC * (_SPATIAL // _BLK_S)
_BLK_PER_W = _SC_BLOCKS // _NW
assert _SC_BLOCKS % _NW == 0 and _BLK_PER_W % 2 == 0

_CH = np.linspace(0, _CDIM - 1, _TDIM).astype(np.int32)


def _issue_block(feat_hbm, blk, buf, sem):
    pltpu.async_copy(
        feat_hbm.at[pl.ds(blk * _BLK_S, _BLK_S), :],
        buf,
        sem,
    )


def _drain_block(feat_hbm, buf, sem):
    pltpu.make_async_copy(
        feat_hbm.at[pl.ds(0, _BLK_S), :],
        buf,
        sem,
    ).wait()


def _compute_block(ch_v, buf, out_v):
    for h in range(2):
        chv = ch_v[h]
        for c_col in range(_PS):
            acc = None
            for r in (2 * c_col, 2 * c_col + 1,
                      _SS + 2 * c_col, _SS + 2 * c_col + 1):
                rv = jnp.full((16,), r, jnp.int32)
                v = plsc.load_gather(buf, [rv, chv])
                acc = v if acc is None else acc + v
            out_v[c_col, pl.ds(h * 16, 16)] = acc * jnp.float32(0.25)


def _store_block(out_hbm, blk, out_v):
    b = blk // (_SPATIAL // _BLK_S)
    g = blk % (_SPATIAL // _BLK_S)
    pltpu.sync_copy(out_v, out_hbm.at[b, pl.ds(g * _PS, _PS)])


def _sc_body(feat_hbm, ch_hbm, out_hbm, ch_v, buf_a, buf_b, out_v,
             sem_a, sem_b):
    wid = lax.axis_index("s") * _NC + lax.axis_index("c")
    pltpu.sync_copy(ch_hbm, ch_v)

    blk0 = wid * _BLK_PER_W
    _issue_block(feat_hbm, blk0, buf_a, sem_a)

    def blk_body(g, _):
        blk = blk0 + g * 2
        _issue_block(feat_hbm, blk + 1, buf_b, sem_b)
        _drain_block(feat_hbm, buf_a, sem_a)
        _compute_block(ch_v, buf_a, out_v)
        _store_block(out_hbm, blk, out_v)

        @pl.when(g < _BLK_PER_W // 2 - 1)
        def _prefetch():
            _issue_block(feat_hbm, blk + 2, buf_a, sem_a)

        _drain_block(feat_hbm, buf_b, sem_b)
        _compute_block(ch_v, buf_b, out_v)
        _store_block(out_hbm, blk + 1, out_v)
        return _

    lax.fori_loop(0, _BLK_PER_W // 2, blk_body, None)


def _sc_part(features):
    feat2d = features.reshape(_B * _SPATIAL, _CDIM)
    chans = jnp.asarray(_CH.reshape(2, 16))
    sc_call = functools.partial(
        pl.kernel,
        mesh=plsc.VectorSubcoreMesh(core_axis_name="c", subcore_axis_name="s"),
        compiler_params=pltpu.CompilerParams(
            use_tc_tiling_on_sc=True, needs_layout_passes=False),
        out_type=jax.ShapeDtypeStruct((_K_SC, _NPOOL, _TDIM), jnp.float32),
        scratch_types=[
            pltpu.VMEM((2, 16), jnp.int32),
            pltpu.VMEM((_BLK_S, _CDIM), jnp.float32),
            pltpu.VMEM((_BLK_S, _CDIM), jnp.float32),
            pltpu.VMEM((_PS, _TDIM), jnp.float32),
            pltpu.SemaphoreType.DMA,
            pltpu.SemaphoreType.DMA,
        ],
    )(_sc_body)
    return sc_call(feat2d, chans)


def _pool_matrix() -> np.ndarray:
    P = np.zeros((_NPOOL, _SS * _SS), dtype=np.float32)
    for R in range(_PS):
        for C in range(_PS):
            p = R * _PS + C
            for dr in range(2):
                for dc in range(2):
                    s = (2 * R + dr) * _SS + (2 * C + dc)
                    P[p, s] = 0.25
    return P


def _select_matrix() -> np.ndarray:
    S = np.zeros((_CDIM, _TDIM), dtype=np.float32)
    S[_CH, np.arange(_TDIM)] = 1.0
    return S


def _tc_body(x_ref, p_ref, s_ref, o_ref):
    x = x_ref[0]
    sel = jnp.dot(x, s_ref[...], preferred_element_type=jnp.float32)
    o_ref[0] = jnp.dot(p_ref[...], sel, preferred_element_type=jnp.float32)


def _tc_part(features):
    nb = _B - _K_SC
    P = jnp.asarray(_pool_matrix())
    S = jnp.asarray(_select_matrix())
    return pl.pallas_call(
        _tc_body,
        grid=(nb,),
        in_specs=[
            pl.BlockSpec((1, _SPATIAL, _CDIM), lambda i: (i + _K_SC, 0, 0)),
            pl.BlockSpec((_NPOOL, _SPATIAL), lambda i: (0, 0)),
            pl.BlockSpec((_CDIM, _TDIM), lambda i: (0, 0)),
        ],
        out_specs=pl.BlockSpec((1, _NPOOL, _TDIM), lambda i: (i, 0, 0)),
        out_shape=jax.ShapeDtypeStruct((nb, _NPOOL, _TDIM), jnp.float32),
    )(features, P, S)


def kernel(features):
    sc_out = _sc_part(features)
    tc_out = _tc_part(features)
    return jnp.concatenate([sc_out, tc_out], axis=0)
